# baseline (speedup 1.0000x reference)
import functools

import jax
import jax.numpy as jnp
from jax import lax
from jax.experimental import pallas as pl
from jax.experimental.pallas import tpu as pltpu
from jax.experimental.pallas import tpu_sc as plsc

B = 16384
H = 50
D = 32
NUM_WORD = 1000000
W = 34

NC = 2
NS = 16
NW = NC * NS
BPW = B // NW
CH = 32
NCHUNK = BPW // CH
IDS_PER_CHUNK = CH * H
GLEN = 100
NGATHER = IDS_PER_CHUNK // GLEN
IDROWS = B * H // GLEN
IDROWS_PER_CHUNK = NGATHER
IDROWS_PER_W = IDROWS // NW

CBLK = 512
OBLK = CBLK * W
NBB = 1952
NBBW = NBB // NW
EX_C0 = NBB * CBLK
TAIL_R0 = 999936
TAIL_N = NUM_WORD - TAIL_R0
TOUT = NUM_WORD * W


def _tr_body(tt_hbm, tailp_hbm, t33_hbm, in0, in1, out0, out1,
             si0, si1, so0, so1):
    wid = lax.axis_index("s") * NC + lax.axis_index("c")
    iota = lax.broadcasted_iota(jnp.int32, (16,), 0)
    pat = iota * W

    def src(j):
        return tt_hbm.at[:, pl.ds(j * CBLK, CBLK)]

    def dst(j):
        return t33_hbm.at[pl.ds(j * OBLK, OBLK)]

    def transpose_buf(inb, outb, ngroups):
        @plsc.parallel_loop(0, D, unroll=4)
        def col(c):
            for g in range(ngroups):
                v = inb[c, pl.ds(g * 16, 16)]
                plsc.store_scatter(outb, [pat + (g * 16 * W + c)], v)

    bufs = ((in0, out0, si0, so0), (in1, out1, si1, so1))

    pltpu.async_copy(src(wid), in0, si0)
    pltpu.async_copy(src(NW + wid), in1, si1)

    def pair(m, carry):
        for par in (0, 1):
            inb, outb, semi, semo = bufs[par]
            k = 2 * m + par
            j = k * NW + wid
            pltpu.make_async_copy(src(j), inb, semi).wait()

            @pl.when(m > 0)
            def _():
                pltpu.make_async_copy(
                    outb, dst((k - 2) * NW + wid), semo).wait()

            transpose_buf(inb, outb, CBLK // 16)

            @pl.when(k + 2 < NBBW)
            def _():
                pltpu.async_copy(src((k + 2) * NW + wid), inb, semi)

            pltpu.async_copy(outb, dst(j), semo)
        return carry

    lax.fori_loop(0, (NBBW - 1) // 2, pair, 0)

    k_last = NBBW - 1
    j_last = k_last * NW + wid
    pltpu.make_async_copy(src(j_last), in0, si0).wait()
    pltpu.make_async_copy(out0, dst((k_last - 2) * NW + wid), so0).wait()
    transpose_buf(in0, out0, CBLK // 16)
    pltpu.async_copy(out0, dst(j_last), so0)
    pltpu.make_async_copy(out1, dst((k_last - 1) * NW + wid), so1).wait()

    @pl.when(wid < 4)
    def _extra():
        c0 = EX_C0 + wid * 128
        pltpu.sync_copy(tt_hbm.at[:, pl.ds(c0, 128)], in1.at[:, pl.ds(0, 128)])
        transpose_buf(in1, out1, 8)
        pltpu.sync_copy(out1.at[pl.ds(0, 128 * W)],
                        t33_hbm.at[pl.ds(c0 * W, 128 * W)])

    @pl.when(wid == 4)
    def _tail():
        pltpu.sync_copy(tailp_hbm, in1.at[:, pl.ds(0, 128)])
        transpose_buf(in1, out1, TAIL_N // 16)
        pltpu.sync_copy(out1.at[pl.ds(0, TAIL_N * W)],
                        t33_hbm.at[pl.ds(TAIL_R0 * W, TAIL_N * W)])

    pltpu.make_async_copy(out0, dst(j_last), so0).wait()


_transpose = functools.partial(
    pl.kernel,
    out_type=jax.ShapeDtypeStruct((TOUT,), jnp.float32),
    mesh=plsc.VectorSubcoreMesh(core_axis_name="c", subcore_axis_name="s"),
    scratch_types=[
        pltpu.VMEM((D, CBLK), jnp.float32),
        pltpu.VMEM((D, CBLK), jnp.float32),
        pltpu.VMEM((OBLK,), jnp.float32),
        pltpu.VMEM((OBLK,), jnp.float32),
        pltpu.SemaphoreType.DMA,
        pltpu.SemaphoreType.DMA,
        pltpu.SemaphoreType.DMA,
        pltpu.SemaphoreType.DMA,
    ],
    compiler_params=pltpu.CompilerParams(
        use_tc_tiling_on_sc=True, needs_layout_passes=False
    ),
)(_tr_body)


def _sc_body(ids_hbm, table_hbm, avg_hbm, idx_v, rows_v, out_v, sem):
    wid = lax.axis_index("s") * NC + lax.axis_index("c")

    def chunk(c, carry):
        idrow0 = wid * IDROWS_PER_W + c * IDROWS_PER_CHUNK
        pltpu.sync_copy(ids_hbm.at[pl.ds(idrow0, IDROWS_PER_CHUNK)], idx_v)
        handles = []
        for g in range(NGATHER):
            handles.append(
                pltpu.async_copy(
                    table_hbm.at[idx_v.at[g]],
                    rows_v.at[pl.ds(g * GLEN, GLEN)],
                    sem,
                )
            )
        for h in handles:
            h.wait()

        def elem(b, carry2):
            base = b * H

            def half(hf):
                col = pl.ds(hf * 16, 16)
                acc = [rows_v[base + k, col] for k in range(4)]
                for k in range(4, H):
                    acc[k % 4] = acc[k % 4] + rows_v[base + k, col]
                return ((acc[0] + acc[1]) + (acc[2] + acc[3])) * (1.0 / H)

            out_v[b, pl.ds(0, 16)] = half(0)
            out_v[b, pl.ds(16, 16)] = half(1)
            return carry2

        lax.fori_loop(0, CH, elem, 0)
        pltpu.sync_copy(out_v, avg_hbm.at[pl.ds(wid * BPW + c * CH, CH)])
        return carry

    lax.fori_loop(0, NCHUNK, chunk, 0)


_sc_avg = functools.partial(
    pl.kernel,
    out_type=jax.ShapeDtypeStruct((B, D), jnp.float32),
    mesh=plsc.VectorSubcoreMesh(core_axis_name="c", subcore_axis_name="s"),
    scratch_types=[
        pltpu.VMEM((IDROWS_PER_CHUNK, GLEN), jnp.int32),
        pltpu.VMEM((IDS_PER_CHUNK, W), jnp.float32),
        pltpu.VMEM((CH, D), jnp.float32),
        pltpu.SemaphoreType.DMA,
    ],
    compiler_params=pltpu.CompilerParams(use_tc_tiling_on_sc=False),
)(_sc_body)


def _ln_body(x_ref, g_ref, b_ref, o_ref):
    x = x_ref[...]
    mu = jnp.mean(x, axis=-1, keepdims=True)
    d = x - mu
    var = jnp.mean(d * d, axis=-1, keepdims=True)
    o_ref[...] = d * lax.rsqrt(var + 1e-5) * g_ref[...] + b_ref[...]


_layernorm = pl.pallas_call(
    _ln_body,
    out_shape=jax.ShapeDtypeStruct((B, D), jnp.float32),
)


def kernel(word_ids, table, gamma, beta):
    ids = word_ids.reshape(IDROWS, GLEN).astype(jnp.int32)
    tailp = jnp.pad(jnp.transpose(table[TAIL_R0:]), ((0, 0), (0, 128 - TAIL_N)))
    t33 = _transpose(jnp.transpose(table), tailp)
    avg = _sc_avg(ids, t33.reshape(NUM_WORD, W))
    return _layernorm(avg, gamma.reshape(1, D), beta.reshape(1, D))


# speedup vs baseline: 6.7547x; 6.7547x over previous
import functools

import jax
import jax.numpy as jnp
from jax import lax
from jax.experimental import pallas as pl
from jax.experimental.pallas import tpu as pltpu
from jax.experimental.pallas import tpu_sc as plsc

B = 16384
H = 50
D = 32
NUM_WORD = 1000000
W = 32
WS = 33

NC = 2
NS = 16
NW = NC * NS
BPW = B // NW
CH = 32
NCHUNK = BPW // CH
IDS_PER_CHUNK = CH * H
GLEN = 100
NGATHER = IDS_PER_CHUNK // GLEN
IDROWS = B * H // GLEN
IDROWS_PER_CHUNK = NGATHER
IDROWS_PER_W = IDROWS // NW

CBLK = 512
OBLK = CBLK * W
NBB = 1952
NBBW = NBB // NW
EX_C0 = NBB * CBLK
TAIL_R0 = 999936
TAIL_N = NUM_WORD - TAIL_R0
TOUT = NUM_WORD * W


def _tr_body(tt_hbm, tailp_hbm, t33_hbm, in0, in1, st0, st1, out0, out1,
             si0, si1, so0, so1):
    wid = lax.axis_index("s") * NC + lax.axis_index("c")
    iota = lax.broadcasted_iota(jnp.int32, (16,), 0)
    pat = iota * WS

    def src(j):
        return tt_hbm.at[:, pl.ds(j * CBLK, CBLK)]

    def dst(j):
        return t33_hbm.at[pl.ds(j * OBLK, OBLK)]

    def transpose_buf(inb, stb, outb, ngroups):
        @plsc.parallel_loop(0, D, unroll=4)
        def col(c):
            for g in range(ngroups):
                v = inb[c, pl.ds(g * 16, 16)]
                plsc.store_scatter(stb, [pat + (g * 16 * WS + c)], v)

        @plsc.parallel_loop(0, ngroups * 16, unroll=4)
        def row(r):
            for hf in (0, 1):
                v = plsc.load_gather(stb, [iota + (r * WS + 16 * hf)])
                outb[pl.ds(r * W + 16 * hf, 16)] = v

    bufs = ((in0, st0, out0, si0, so0), (in1, st1, out1, si1, so1))

    pltpu.async_copy(src(wid), in0, si0)
    pltpu.async_copy(src(NW + wid), in1, si1)

    def pair(m, carry):
        for par in (0, 1):
            inb, stb, outb, semi, semo = bufs[par]
            k = 2 * m + par
            j = k * NW + wid
            pltpu.make_async_copy(src(j), inb, semi).wait()

            @pl.when(m > 0)
            def _():
                pltpu.make_async_copy(
                    outb, dst((k - 2) * NW + wid), semo).wait()

            transpose_buf(inb, stb, outb, CBLK // 16)

            @pl.when(k + 2 < NBBW)
            def _():
                pltpu.async_copy(src((k + 2) * NW + wid), inb, semi)

            pltpu.async_copy(outb, dst(j), semo)
        return carry

    lax.fori_loop(0, (NBBW - 1) // 2, pair, 0)

    k_last = NBBW - 1
    j_last = k_last * NW + wid
    pltpu.make_async_copy(src(j_last), in0, si0).wait()
    pltpu.make_async_copy(out0, dst((k_last - 2) * NW + wid), so0).wait()
    transpose_buf(in0, st0, out0, CBLK // 16)
    pltpu.async_copy(out0, dst(j_last), so0)
    pltpu.make_async_copy(out1, dst((k_last - 1) * NW + wid), so1).wait()

    @pl.when(wid < 4)
    def _extra():
        c0 = EX_C0 + wid * 128
        pltpu.sync_copy(tt_hbm.at[:, pl.ds(c0, 128)], in1.at[:, pl.ds(0, 128)])
        transpose_buf(in1, st1, out1, 8)
        pltpu.sync_copy(out1.at[pl.ds(0, 128 * W)],
                        t33_hbm.at[pl.ds(c0 * W, 128 * W)])

    @pl.when(wid == 4)
    def _tail():
        pltpu.sync_copy(tailp_hbm, in1.at[:, pl.ds(0, 128)])
        transpose_buf(in1, st1, out1, TAIL_N // 16)
        pltpu.sync_copy(out1.at[pl.ds(0, TAIL_N * W)],
                        t33_hbm.at[pl.ds(TAIL_R0 * W, TAIL_N * W)])

    pltpu.make_async_copy(out0, dst(j_last), so0).wait()


_transpose = functools.partial(
    pl.kernel,
    out_type=jax.ShapeDtypeStruct((TOUT,), jnp.float32),
    mesh=plsc.VectorSubcoreMesh(core_axis_name="c", subcore_axis_name="s"),
    scratch_types=[
        pltpu.VMEM((D, CBLK), jnp.float32),
        pltpu.VMEM((D, CBLK), jnp.float32),
        pltpu.VMEM((CBLK * WS,), jnp.float32),
        pltpu.VMEM((CBLK * WS,), jnp.float32),
        pltpu.VMEM((OBLK,), jnp.float32),
        pltpu.VMEM((OBLK,), jnp.float32),
        pltpu.SemaphoreType.DMA,
        pltpu.SemaphoreType.DMA,
        pltpu.SemaphoreType.DMA,
        pltpu.SemaphoreType.DMA,
    ],
    compiler_params=pltpu.CompilerParams(
        use_tc_tiling_on_sc=True, needs_layout_passes=False
    ),
)(_tr_body)


def _sc_body(ids_hbm, table_hbm, avg_hbm, idx_v, rows_v, out_v, sem):
    wid = lax.axis_index("s") * NC + lax.axis_index("c")

    def chunk(c, carry):
        idrow0 = wid * IDROWS_PER_W + c * IDROWS_PER_CHUNK
        pltpu.sync_copy(ids_hbm.at[pl.ds(idrow0, IDROWS_PER_CHUNK)], idx_v)
        handles = []
        for g in range(NGATHER):
            handles.append(
                pltpu.async_copy(
                    table_hbm.at[idx_v.at[g]],
                    rows_v.at[pl.ds(g * GLEN, GLEN)],
                    sem,
                )
            )
        for h in handles:
            h.wait()

        def elem(b, carry2):
            base = b * H

            def half(hf):
                col = pl.ds(hf * 16, 16)
                acc = [rows_v[base + k, col] for k in range(4)]
                for k in range(4, H):
                    acc[k % 4] = acc[k % 4] + rows_v[base + k, col]
                return ((acc[0] + acc[1]) + (acc[2] + acc[3])) * (1.0 / H)

            out_v[b, pl.ds(0, 16)] = half(0)
            out_v[b, pl.ds(16, 16)] = half(1)
            return carry2

        lax.fori_loop(0, CH, elem, 0)
        pltpu.sync_copy(out_v, avg_hbm.at[pl.ds(wid * BPW + c * CH, CH)])
        return carry

    lax.fori_loop(0, NCHUNK, chunk, 0)


_sc_avg = functools.partial(
    pl.kernel,
    out_type=jax.ShapeDtypeStruct((B, D), jnp.float32),
    mesh=plsc.VectorSubcoreMesh(core_axis_name="c", subcore_axis_name="s"),
    scratch_types=[
        pltpu.VMEM((IDROWS_PER_CHUNK, GLEN), jnp.int32),
        pltpu.VMEM((IDS_PER_CHUNK, W), jnp.float32),
        pltpu.VMEM((CH, D), jnp.float32),
        pltpu.SemaphoreType.DMA,
    ],
    compiler_params=pltpu.CompilerParams(use_tc_tiling_on_sc=False),
)(_sc_body)


def _ln_body(x_ref, g_ref, b_ref, o_ref):
    x = x_ref[...]
    mu = jnp.mean(x, axis=-1, keepdims=True)
    d = x - mu
    var = jnp.mean(d * d, axis=-1, keepdims=True)
    o_ref[...] = d * lax.rsqrt(var + 1e-5) * g_ref[...] + b_ref[...]


_layernorm = pl.pallas_call(
    _ln_body,
    out_shape=jax.ShapeDtypeStruct((B, D), jnp.float32),
)


def kernel(word_ids, table, gamma, beta):
    ids = word_ids.reshape(IDROWS, GLEN).astype(jnp.int32)
    tailp = jnp.pad(jnp.transpose(table[TAIL_R0:]), ((0, 0), (0, 128 - TAIL_N)))
    t33 = _transpose(jnp.transpose(table), tailp)
    avg = _sc_avg(ids, t33.reshape(NUM_WORD, W))
    return _layernorm(avg, gamma.reshape(1, D), beta.reshape(1, D))


# speedup vs baseline: 7.7462x; 1.1468x over previous
import functools

import jax
import jax.numpy as jnp
from jax import lax
from jax.experimental import pallas as pl
from jax.experimental.pallas import tpu as pltpu
from jax.experimental.pallas import tpu_sc as plsc

B = 16384
H = 50
D = 32
NUM_WORD = 1000000
W = 32
WS = 33

NC = 2
NS = 16
NW = NC * NS
BPW = B // NW
CH = 32
NCHUNK = BPW // CH
IDS_PER_CHUNK = CH * H
GLEN = 100
NGATHER = IDS_PER_CHUNK // GLEN
IDROWS = B * H // GLEN
IDROWS_PER_CHUNK = NGATHER
IDROWS_PER_W = IDROWS // NW

CBLK = 512
OBLK = CBLK * W
NBB = 1952
NBBW = NBB // NW
EX_C0 = NBB * CBLK
TAIL_R0 = 999936
TAIL_N = NUM_WORD - TAIL_R0
TOUT = NUM_WORD * W


def _tr_body(tt_hbm, tailp_hbm, t33_hbm, in0, in1, st0, st1, out0, out1,
             si0, si1, so0, so1):
    wid = lax.axis_index("s") * NC + lax.axis_index("c")
    iota = lax.broadcasted_iota(jnp.int32, (16,), 0)
    pat = iota * WS

    def src(j):
        return tt_hbm.at[:, pl.ds(j * CBLK, CBLK)]

    def dst(j):
        return t33_hbm.at[pl.ds(j * OBLK, OBLK)]

    def transpose_buf(inb, stb, outb, ngroups):
        @plsc.parallel_loop(0, D, unroll=4)
        def col(c):
            for g in range(ngroups):
                v = inb[c, pl.ds(g * 16, 16)]
                plsc.store_scatter(stb, [pat + (g * 16 * WS + c)], v)

        @plsc.parallel_loop(0, ngroups * 16, unroll=4)
        def row(r):
            for hf in (0, 1):
                v = plsc.load_gather(stb, [iota + (r * WS + 16 * hf)])
                outb[pl.ds(r * W + 16 * hf, 16)] = v

    bufs = ((in0, st0, out0, si0, so0), (in1, st1, out1, si1, so1))

    pltpu.async_copy(src(wid), in0, si0)
    pltpu.async_copy(src(NW + wid), in1, si1)

    def pair(m, carry):
        for par in (0, 1):
            inb, stb, outb, semi, semo = bufs[par]
            k = 2 * m + par
            j = k * NW + wid
            pltpu.make_async_copy(src(j), inb, semi).wait()

            @pl.when(m > 0)
            def _():
                pltpu.make_async_copy(
                    outb, dst((k - 2) * NW + wid), semo).wait()

            transpose_buf(inb, stb, outb, CBLK // 16)

            @pl.when(k + 2 < NBBW)
            def _():
                pltpu.async_copy(src((k + 2) * NW + wid), inb, semi)

            pltpu.async_copy(outb, dst(j), semo)
        return carry

    lax.fori_loop(0, (NBBW - 1) // 2, pair, 0)

    k_last = NBBW - 1
    j_last = k_last * NW + wid
    pltpu.make_async_copy(src(j_last), in0, si0).wait()
    pltpu.make_async_copy(out0, dst((k_last - 2) * NW + wid), so0).wait()
    transpose_buf(in0, st0, out0, CBLK // 16)
    pltpu.async_copy(out0, dst(j_last), so0)
    pltpu.make_async_copy(out1, dst((k_last - 1) * NW + wid), so1).wait()

    @pl.when(wid < 4)
    def _extra():
        c0 = EX_C0 + wid * 128
        pltpu.sync_copy(tt_hbm.at[:, pl.ds(c0, 128)], in1.at[:, pl.ds(0, 128)])
        transpose_buf(in1, st1, out1, 8)
        pltpu.sync_copy(out1.at[pl.ds(0, 128 * W)],
                        t33_hbm.at[pl.ds(c0 * W, 128 * W)])

    @pl.when(wid == 4)
    def _tail():
        pltpu.sync_copy(tailp_hbm, in1.at[:, pl.ds(0, 128)])
        transpose_buf(in1, st1, out1, TAIL_N // 16)
        pltpu.sync_copy(out1.at[pl.ds(0, TAIL_N * W)],
                        t33_hbm.at[pl.ds(TAIL_R0 * W, TAIL_N * W)])

    pltpu.make_async_copy(out0, dst(j_last), so0).wait()


_transpose = functools.partial(
    pl.kernel,
    out_type=jax.ShapeDtypeStruct((TOUT,), jnp.float32),
    mesh=plsc.VectorSubcoreMesh(core_axis_name="c", subcore_axis_name="s"),
    scratch_types=[
        pltpu.VMEM((D, CBLK), jnp.float32),
        pltpu.VMEM((D, CBLK), jnp.float32),
        pltpu.VMEM((CBLK * WS,), jnp.float32),
        pltpu.VMEM((CBLK * WS,), jnp.float32),
        pltpu.VMEM((OBLK,), jnp.float32),
        pltpu.VMEM((OBLK,), jnp.float32),
        pltpu.SemaphoreType.DMA,
        pltpu.SemaphoreType.DMA,
        pltpu.SemaphoreType.DMA,
        pltpu.SemaphoreType.DMA,
    ],
    compiler_params=pltpu.CompilerParams(
        use_tc_tiling_on_sc=True, needs_layout_passes=False
    ),
)(_tr_body)


def _sc_body(ids_hbm, table_hbm, avg_hbm, idx0, idx1, rows0, rows1, out_v,
             sg0, sg1):
    wid = lax.axis_index("s") * NC + lax.axis_index("c")
    gbufs = ((idx0, rows0, sg0), (idx1, rows1, sg1))

    def load_and_fire(c, par):
        idxb, rowsb, sem = gbufs[par]
        idrow0 = wid * IDROWS_PER_W + c * IDROWS_PER_CHUNK
        pltpu.sync_copy(ids_hbm.at[pl.ds(idrow0, IDROWS_PER_CHUNK)], idxb)
        for g in range(NGATHER):
            pltpu.async_copy(table_hbm.at[idxb.at[g]],
                             rowsb.at[pl.ds(g * GLEN, GLEN)], sem)

    def drain(par):
        idxb, rowsb, sem = gbufs[par]
        for g in range(NGATHER):
            pltpu.make_async_copy(table_hbm.at[idxb.at[g]],
                                  rowsb.at[pl.ds(g * GLEN, GLEN)], sem).wait()

    def compute(c, par):
        rowsb = gbufs[par][1]

        def elem(b, carry2):
            base = b * H

            def half(hf):
                col = pl.ds(hf * 16, 16)
                acc = [rowsb[base + k, col] for k in range(4)]
                for k in range(4, H):
                    acc[k % 4] = acc[k % 4] + rowsb[base + k, col]
                return ((acc[0] + acc[1]) + (acc[2] + acc[3])) * (1.0 / H)

            out_v[b, pl.ds(0, 16)] = half(0)
            out_v[b, pl.ds(16, 16)] = half(1)
            return carry2

        lax.fori_loop(0, CH, elem, 0)
        pltpu.sync_copy(out_v, avg_hbm.at[pl.ds(wid * BPW + c * CH, CH)])

    load_and_fire(0, 0)
    load_and_fire(1, 1)

    def piped(m, carry):
        for par in (0, 1):
            c = 2 * m + par
            drain(par)
            compute(c, par)
            load_and_fire(c + 2, par)
        return carry

    lax.fori_loop(0, NCHUNK // 2 - 1, piped, 0)
    drain(0)
    compute(NCHUNK - 2, 0)
    drain(1)
    compute(NCHUNK - 1, 1)


_sc_avg = functools.partial(
    pl.kernel,
    out_type=jax.ShapeDtypeStruct((B, D), jnp.float32),
    mesh=plsc.VectorSubcoreMesh(core_axis_name="c", subcore_axis_name="s"),
    scratch_types=[
        pltpu.VMEM((IDROWS_PER_CHUNK, GLEN), jnp.int32),
        pltpu.VMEM((IDROWS_PER_CHUNK, GLEN), jnp.int32),
        pltpu.VMEM((IDS_PER_CHUNK, W), jnp.float32),
        pltpu.VMEM((IDS_PER_CHUNK, W), jnp.float32),
        pltpu.VMEM((CH, D), jnp.float32),
        pltpu.SemaphoreType.DMA,
        pltpu.SemaphoreType.DMA,
    ],
    compiler_params=pltpu.CompilerParams(use_tc_tiling_on_sc=False),
)(_sc_body)


def _ln_body(x_ref, g_ref, b_ref, o_ref):
    x = x_ref[...]
    mu = jnp.mean(x, axis=-1, keepdims=True)
    d = x - mu
    var = jnp.mean(d * d, axis=-1, keepdims=True)
    o_ref[...] = d * lax.rsqrt(var + 1e-5) * g_ref[...] + b_ref[...]


_layernorm = pl.pallas_call(
    _ln_body,
    out_shape=jax.ShapeDtypeStruct((B, D), jnp.float32),
)


def kernel(word_ids, table, gamma, beta):
    ids = word_ids.reshape(IDROWS, GLEN).astype(jnp.int32)
    tailp = jnp.pad(jnp.transpose(table[TAIL_R0:]), ((0, 0), (0, 128 - TAIL_N)))
    t33 = _transpose(jnp.transpose(table), tailp)
    avg = _sc_avg(ids, t33.reshape(NUM_WORD, W))
    return _layernorm(avg, gamma.reshape(1, D), beta.reshape(1, D))


# speedup vs baseline: 7.8787x; 1.0171x over previous
import functools

import jax
import jax.numpy as jnp
from jax import lax
from jax.experimental import pallas as pl
from jax.experimental.pallas import tpu as pltpu
from jax.experimental.pallas import tpu_sc as plsc

B = 16384
H = 50
D = 32
NUM_WORD = 1000000
W = 32
WS = 33

NC = 2
NS = 16
NW = NC * NS
BPW = B // NW
CH = 32
NCHUNK = BPW // CH
IDS_PER_CHUNK = CH * H
GLEN = 100
NGATHER = IDS_PER_CHUNK // GLEN
IDROWS = B * H // GLEN
IDROWS_PER_CHUNK = NGATHER
IDROWS_PER_W = IDROWS // NW

CBLK = 512
OBLK = CBLK * W
NBB = 1952
NBBW = NBB // NW
EX_C0 = NBB * CBLK
TAIL_R0 = 999936
TAIL_N = NUM_WORD - TAIL_R0
TOUT = NUM_WORD * W


def _tr_body(tt_hbm, tailp_hbm, t33_hbm, in0, in1, st0, st1, out0, out1,
             si0, si1, so0, so1):
    wid = lax.axis_index("s") * NC + lax.axis_index("c")
    iota = lax.broadcasted_iota(jnp.int32, (16,), 0)
    pat = iota * WS

    def src(j):
        return tt_hbm.at[:, pl.ds(j * CBLK, CBLK)]

    def dst(j):
        return t33_hbm.at[pl.ds(j * OBLK, OBLK)]

    def transpose_buf(inb, stb, outb, ngroups):
        @plsc.parallel_loop(0, D, unroll=4)
        def col(c):
            for g in range(ngroups):
                v = inb[c, pl.ds(g * 16, 16)]
                plsc.store_scatter(stb, [pat + (g * 16 * WS + c)], v)

        @plsc.parallel_loop(0, ngroups * 16, unroll=4)
        def row(r):
            for hf in (0, 1):
                v = plsc.load_gather(stb, [iota + (r * WS + 16 * hf)])
                outb[pl.ds(r * W + 16 * hf, 16)] = v

    bufs = ((in0, st0, out0, si0, so0), (in1, st1, out1, si1, so1))

    pltpu.async_copy(src(wid), in0, si0)
    pltpu.async_copy(src(NW + wid), in1, si1)

    def pair(m, carry):
        for par in (0, 1):
            inb, stb, outb, semi, semo = bufs[par]
            k = 2 * m + par
            j = k * NW + wid
            pltpu.make_async_copy(src(j), inb, semi).wait()

            @pl.when(m > 0)
            def _():
                pltpu.make_async_copy(
                    outb, dst((k - 2) * NW + wid), semo).wait()

            transpose_buf(inb, stb, outb, CBLK // 16)

            @pl.when(k + 2 < NBBW)
            def _():
                pltpu.async_copy(src((k + 2) * NW + wid), inb, semi)

            pltpu.async_copy(outb, dst(j), semo)
        return carry

    lax.fori_loop(0, (NBBW - 1) // 2, pair, 0)

    k_last = NBBW - 1
    j_last = k_last * NW + wid
    pltpu.make_async_copy(src(j_last), in0, si0).wait()
    pltpu.make_async_copy(out0, dst((k_last - 2) * NW + wid), so0).wait()
    transpose_buf(in0, st0, out0, CBLK // 16)
    pltpu.async_copy(out0, dst(j_last), so0)
    pltpu.make_async_copy(out1, dst((k_last - 1) * NW + wid), so1).wait()

    @pl.when(wid < 4)
    def _extra():
        c0 = EX_C0 + wid * 128
        pltpu.sync_copy(tt_hbm.at[:, pl.ds(c0, 128)], in1.at[:, pl.ds(0, 128)])
        transpose_buf(in1, st1, out1, 8)
        pltpu.sync_copy(out1.at[pl.ds(0, 128 * W)],
                        t33_hbm.at[pl.ds(c0 * W, 128 * W)])

    @pl.when(wid == 4)
    def _tail():
        pltpu.sync_copy(tailp_hbm, in1.at[:, pl.ds(0, 128)])
        transpose_buf(in1, st1, out1, TAIL_N // 16)
        pltpu.sync_copy(out1.at[pl.ds(0, TAIL_N * W)],
                        t33_hbm.at[pl.ds(TAIL_R0 * W, TAIL_N * W)])

    pltpu.make_async_copy(out0, dst(j_last), so0).wait()


_transpose = functools.partial(
    pl.kernel,
    out_type=jax.ShapeDtypeStruct((TOUT,), jnp.float32),
    mesh=plsc.VectorSubcoreMesh(core_axis_name="c", subcore_axis_name="s"),
    scratch_types=[
        pltpu.VMEM((D, CBLK), jnp.float32),
        pltpu.VMEM((D, CBLK), jnp.float32),
        pltpu.VMEM((CBLK * WS,), jnp.float32),
        pltpu.VMEM((CBLK * WS,), jnp.float32),
        pltpu.VMEM((OBLK,), jnp.float32),
        pltpu.VMEM((OBLK,), jnp.float32),
        pltpu.SemaphoreType.DMA,
        pltpu.SemaphoreType.DMA,
        pltpu.SemaphoreType.DMA,
        pltpu.SemaphoreType.DMA,
    ],
    compiler_params=pltpu.CompilerParams(
        use_tc_tiling_on_sc=True, needs_layout_passes=False
    ),
)(_tr_body)


def _sc_body(ids_hbm, table_hbm, avg_hbm, idx0, idx1, rows0, rows1, out_v,
             sg0, sg1):
    wid = lax.axis_index("s") * NC + lax.axis_index("c")
    gbufs = ((idx0, rows0, sg0), (idx1, rows1, sg1))

    def load_and_fire(c, par):
        idxb, rowsb, sem = gbufs[par]
        idrow0 = wid * IDROWS_PER_W + c * IDROWS_PER_CHUNK
        pltpu.sync_copy(ids_hbm.at[pl.ds(idrow0, IDROWS_PER_CHUNK)], idxb)
        for g in range(NGATHER):
            pltpu.async_copy(table_hbm.at[idxb.at[g]],
                             rowsb.at[pl.ds(g * GLEN, GLEN)], sem)

    def drain(par):
        idxb, rowsb, sem = gbufs[par]
        for g in range(NGATHER):
            pltpu.make_async_copy(table_hbm.at[idxb.at[g]],
                                  rowsb.at[pl.ds(g * GLEN, GLEN)], sem).wait()

    def compute(c, par):
        rowsb = gbufs[par][1]

        @plsc.parallel_loop(0, CH, unroll=2)
        def elem(b):
            base = b * H

            def half(hf):
                col = pl.ds(hf * 16, 16)
                acc = [rowsb[base + k, col] for k in range(4)]
                for k in range(4, H):
                    acc[k % 4] = acc[k % 4] + rowsb[base + k, col]
                return ((acc[0] + acc[1]) + (acc[2] + acc[3])) * (1.0 / H)

            out_v[b, pl.ds(0, 16)] = half(0)
            out_v[b, pl.ds(16, 16)] = half(1)
        pltpu.sync_copy(out_v, avg_hbm.at[pl.ds(wid * BPW + c * CH, CH)])

    load_and_fire(0, 0)
    load_and_fire(1, 1)

    def piped(m, carry):
        for par in (0, 1):
            c = 2 * m + par
            drain(par)
            compute(c, par)
            load_and_fire(c + 2, par)
        return carry

    lax.fori_loop(0, NCHUNK // 2 - 1, piped, 0)
    drain(0)
    compute(NCHUNK - 2, 0)
    drain(1)
    compute(NCHUNK - 1, 1)


_sc_avg = functools.partial(
    pl.kernel,
    out_type=jax.ShapeDtypeStruct((B, D), jnp.float32),
    mesh=plsc.VectorSubcoreMesh(core_axis_name="c", subcore_axis_name="s"),
    scratch_types=[
        pltpu.VMEM((IDROWS_PER_CHUNK, GLEN), jnp.int32),
        pltpu.VMEM((IDROWS_PER_CHUNK, GLEN), jnp.int32),
        pltpu.VMEM((IDS_PER_CHUNK, W), jnp.float32),
        pltpu.VMEM((IDS_PER_CHUNK, W), jnp.float32),
        pltpu.VMEM((CH, D), jnp.float32),
        pltpu.SemaphoreType.DMA,
        pltpu.SemaphoreType.DMA,
    ],
    compiler_params=pltpu.CompilerParams(use_tc_tiling_on_sc=False),
)(_sc_body)


def _ln_body(x_ref, g_ref, b_ref, o_ref):
    x = x_ref[...]
    mu = jnp.mean(x, axis=-1, keepdims=True)
    d = x - mu
    var = jnp.mean(d * d, axis=-1, keepdims=True)
    o_ref[...] = d * lax.rsqrt(var + 1e-5) * g_ref[...] + b_ref[...]


_layernorm = pl.pallas_call(
    _ln_body,
    out_shape=jax.ShapeDtypeStruct((B, D), jnp.float32),
)


def kernel(word_ids, table, gamma, beta):
    ids = word_ids.reshape(IDROWS, GLEN).astype(jnp.int32)
    tailp = jnp.pad(jnp.transpose(table[TAIL_R0:]), ((0, 0), (0, 128 - TAIL_N)))
    t33 = _transpose(jnp.transpose(table), tailp)
    avg = _sc_avg(ids, t33.reshape(NUM_WORD, W))
    return _layernorm(avg, gamma.reshape(1, D), beta.reshape(1, D))
